# baseline (speedup 1.0000x reference)
import functools

import jax
import jax.numpy as jnp
from jax import lax
from jax.experimental import pallas as pl
from jax.experimental.pallas import tpu as pltpu
from jax.experimental.pallas import tpu_sc as plsc

N = 10000
E = 320000
D = 128

NC = 2
NS = 16
NW = NC * NS
KE = 128
NCHP = 80
EPW = NCHP * KE
E_PAD = NW * EPW
WPB = 8
NWIN = NCHP // WPB
N_PAD = N + 8
NPADE = E_PAD - E
RPS = N // NS

_mesh = plsc.VectorSubcoreMesh(
    core_axis_name="c", subcore_axis_name="s", num_cores=NC, num_subcores=NS
)


def _worker_id():
    return lax.axis_index("s") * NC + lax.axis_index("c")


def _fill(buf, nrows, value):
    @pl.loop(0, nrows)
    def _(r):
        @pl.loop(0, D // 16)
        def _(cc):
            buf[r, pl.ds(cc * 16, 16)] = jnp.full((16,), value, jnp.float32)


def _zero_acc_slice(zbuf, acc_sh, sid):
    @pl.loop(0, RPS // KE)
    def _(j):
        pltpu.sync_copy(zbuf, acc_sh.at[pl.ds(sid * RPS + j * KE, KE)])

    rem = RPS - (RPS // KE) * KE
    pltpu.sync_copy(
        zbuf.at[pl.ds(0, rem)],
        acc_sh.at[pl.ds(sid * RPS + (RPS // KE) * KE, rem)],
    )


@functools.partial(
    pl.kernel,
    out_type=jax.ShapeDtypeStruct((NC, NS, RPS, D), jnp.float32),
    mesh=_mesh,
    scratch_types=[
        pltpu.VMEM((NCHP, KE), jnp.int32),
        pltpu.VMEM((KE, D), jnp.float32),
        pltpu.VMEM_SHARED((N, D), jnp.float32),
        pltpu.SemaphoreType.DMA,
    ],
)
def _hist_kernel(dst_hbm, out_hbm, dst_v, ones_v, acc_sh, sem):
    cid = lax.axis_index("c")
    sid = lax.axis_index("s")
    wid = _worker_id()

    _fill(ones_v, KE, 0.0)
    _zero_acc_slice(ones_v, acc_sh, sid)
    _fill(ones_v, KE, 1.0)

    pltpu.async_copy(dst_hbm.at[wid], dst_v, sem).wait()
    plsc.subcore_barrier()

    @pl.loop(0, NCHP)
    def _(i):
        pltpu.sync_copy(ones_v, acc_sh.at[dst_v.at[i]], add=True)

    plsc.subcore_barrier()
    pltpu.sync_copy(
        acc_sh.at[pl.ds(sid * RPS, RPS)],
        out_hbm.at[cid, sid],
    )


@functools.partial(
    pl.kernel,
    out_type=jax.ShapeDtypeStruct((NC, NS, RPS, D), jnp.float32),
    mesh=_mesh,
    scratch_types=[
        pltpu.VMEM((NCHP, KE), jnp.int32),
        pltpu.VMEM((NCHP, KE), jnp.int32),
        pltpu.VMEM((KE, D), jnp.float32),
        pltpu.VMEM_SHARED((N, D), jnp.float32),
        pltpu.SemaphoreType.DMA,
        pltpu.SemaphoreType.DMA,
    ],
)
def _agg_kernel(h_hbm, src_hbm, dst_hbm, out_hbm,
                src_v, dst_v, rows0, acc_sh, gs0, gs1):
    cid = lax.axis_index("c")
    sid = lax.axis_index("s")
    wid = _worker_id()

    _fill(rows0, KE, 0.0)
    _zero_acc_slice(rows0, acc_sh, sid)

    pltpu.async_copy(src_hbm.at[wid], src_v, gs0)
    pltpu.async_copy(dst_hbm.at[wid], dst_v, gs1)
    pltpu.make_async_copy(src_hbm.at[wid], src_v, gs0).wait()
    pltpu.make_async_copy(dst_hbm.at[wid], dst_v, gs1).wait()
    plsc.subcore_barrier()

    @pl.loop(0, NCHP)
    def _(i):
        pltpu.async_copy(h_hbm.at[src_v.at[i]], rows0, gs0).wait()
        pltpu.sync_copy(rows0, acc_sh.at[dst_v.at[i]], add=True)

    plsc.subcore_barrier()
    pltpu.sync_copy(
        acc_sh.at[pl.ds(sid * RPS, RPS)],
        out_hbm.at[cid, sid],
    )


_RB = 400
_GRID = N // _RB

_row_spec = pl.BlockSpec((_RB, D), lambda i: (i, 0))
_pair_spec = pl.BlockSpec((NC, _RB, D), lambda i: (0, i, 0))
_w_spec = pl.BlockSpec((D, D), lambda i: (0, 0))
_b_spec = pl.BlockSpec((1, D), lambda i: (0, 0))


def _dinv_of(hist_ref):
    i = pl.program_id(0)
    rid = i * _RB + lax.broadcasted_iota(jnp.int32, (_RB, 1), 0)[:, 0]
    pad_cnt = (rid < NPADE).astype(jnp.float32)
    deg = hist_ref[0, :, 0] + hist_ref[1, :, 0] + 1.0 - pad_cnt
    return lax.rsqrt(deg)


def _prep_body(x_ref, w_ref, hist_ref, o_ref):
    dinv = _dinv_of(hist_ref)
    h = jnp.dot(x_ref[...], w_ref[...], preferred_element_type=jnp.float32)
    o_ref[...] = h * dinv[:, None]


_prep = pl.pallas_call(
    _prep_body,
    grid=(_GRID,),
    in_specs=[_row_spec, _w_spec, _pair_spec],
    out_specs=_row_spec,
    out_shape=jax.ShapeDtypeStruct((N, D), jnp.float32),
)


def _mid_body(p_ref, hp_ref, b_ref, w_ref, hist_ref, o_ref):
    dinv = _dinv_of(hist_ref)
    pre = dinv[:, None] * (p_ref[0] + p_ref[1] + hp_ref[...]) + b_ref[...]
    act = jnp.maximum(pre, 0.0)
    h = jnp.dot(act, w_ref[...], preferred_element_type=jnp.float32)
    o_ref[...] = h * dinv[:, None]


_mid = pl.pallas_call(
    _mid_body,
    grid=(_GRID,),
    in_specs=[_pair_spec, _row_spec, _b_spec, _w_spec, _pair_spec],
    out_specs=_row_spec,
    out_shape=jax.ShapeDtypeStruct((N, D), jnp.float32),
)


def _final_body(p_ref, hp_ref, b_ref, hist_ref, o_ref):
    dinv = _dinv_of(hist_ref)
    o_ref[...] = (
        dinv[:, None] * (p_ref[0] + p_ref[1] + hp_ref[...]) + b_ref[...]
    )


_final = pl.pallas_call(
    _final_body,
    grid=(_GRID,),
    in_specs=[_pair_spec, _row_spec, _b_spec, _pair_spec],
    out_specs=_row_spec,
    out_shape=jax.ShapeDtypeStruct((N, D), jnp.float32),
)


def kernel(x, edge_index, W1, b1, W2, b2):
    src = jnp.concatenate(
        [edge_index[0].astype(jnp.int32), jnp.full((NPADE,), N, jnp.int32)]
    ).reshape(NW, NCHP, KE)
    dst = jnp.concatenate(
        [edge_index[1].astype(jnp.int32), jnp.arange(NPADE, dtype=jnp.int32)]
    ).reshape(NW, NCHP, KE)
    b1r = b1.reshape(1, D)
    b2r = b2.reshape(1, D)
    zrows = jnp.zeros((N_PAD - N, D), jnp.float32)

    hist = _hist_kernel(dst).reshape(NC, N, D)

    h1p = _prep(x, W1, hist)
    p = _agg_kernel(jnp.concatenate([h1p, zrows]), src, dst).reshape(NC, N, D)
    h2p = _mid(p, h1p, b1r, W2, hist)
    q = _agg_kernel(jnp.concatenate([h2p, zrows]), src, dst).reshape(NC, N, D)
    out = _final(q, h2p, b2r, hist)
    return out


# speedup vs baseline: 2.6848x; 2.6848x over previous
import functools

import jax
import jax.numpy as jnp
from jax import lax
from jax.experimental import pallas as pl
from jax.experimental.pallas import tpu as pltpu
from jax.experimental.pallas import tpu_sc as plsc

N = 10000
E = 320000
D = 128

NC = 2
NS = 16
NW = NC * NS
KE = 80
NCHP = 125
EPW = NCHP * KE
WPB = 8
FW = NCHP // WPB
TWC = NCHP % WPB
RPS = N // NS

_mesh = plsc.VectorSubcoreMesh(
    core_axis_name="c", subcore_axis_name="s", num_cores=NC, num_subcores=NS
)


def _worker_id():
    return lax.axis_index("s") * NC + lax.axis_index("c")


def _fill(buf, nrows, value):
    @pl.loop(0, nrows)
    def _(r):
        @pl.loop(0, D // 16)
        def _(cc):
            buf[r, pl.ds(cc * 16, 16)] = jnp.full((16,), value, jnp.float32)


def _zero_acc_slice(zbuf, acc_sh, sid):
    @pl.loop(0, RPS // KE)
    def _(j):
        pltpu.sync_copy(zbuf, acc_sh.at[pl.ds(sid * RPS + j * KE, KE)])

    rem = RPS - (RPS // KE) * KE
    if rem:
        pltpu.sync_copy(
            zbuf.at[pl.ds(0, rem)],
            acc_sh.at[pl.ds(sid * RPS + (RPS // KE) * KE, rem)],
        )


@functools.partial(
    pl.kernel,
    out_type=jax.ShapeDtypeStruct((NC, NS, RPS, D), jnp.float32),
    mesh=_mesh,
    scratch_types=[
        pltpu.VMEM((NCHP, KE), jnp.int32),
        pltpu.VMEM((KE, D), jnp.float32),
        pltpu.VMEM_SHARED((N, D), jnp.float32),
        pltpu.SemaphoreType.DMA,
    ],
)
def _hist_kernel(dst_hbm, out_hbm, dst_v, ones_v, acc_sh, sem):
    cid = lax.axis_index("c")
    sid = lax.axis_index("s")
    wid = _worker_id()

    _fill(ones_v, KE, 0.0)
    _zero_acc_slice(ones_v, acc_sh, sid)
    _fill(ones_v, KE, 1.0)

    pltpu.async_copy(dst_hbm.at[wid], dst_v, sem).wait()
    plsc.subcore_barrier()

    @pl.loop(0, NCHP)
    def _(i):
        pltpu.sync_copy(ones_v, acc_sh.at[dst_v.at[i]], add=True)

    plsc.subcore_barrier()
    pltpu.sync_copy(
        acc_sh.at[pl.ds(sid * RPS, RPS)],
        out_hbm.at[cid, sid],
    )


@functools.partial(
    pl.kernel,
    out_type=jax.ShapeDtypeStruct((NC, NS, RPS, D), jnp.float32),
    mesh=_mesh,
    scratch_types=[
        pltpu.VMEM((NCHP, KE), jnp.int32),
        pltpu.VMEM((WPB, KE), jnp.int32),
        pltpu.VMEM((WPB, KE), jnp.int32),
        pltpu.VMEM((KE, D), jnp.float32),
        pltpu.VMEM((KE, D), jnp.float32),
        pltpu.VMEM_SHARED((N, D), jnp.float32),
        pltpu.SemaphoreType.DMA,
        pltpu.SemaphoreType.DMA,
        pltpu.SemaphoreType.DMA,
        pltpu.SemaphoreType.DMA,
    ],
)
def _agg_kernel(h_hbm, src_hbm, dst_hbm, out_hbm,
                src_v, dwa, dwb, rows0, rows1, acc_sh, gs0, gs1, wsa, wsb):
    cid = lax.axis_index("c")
    sid = lax.axis_index("s")
    wid = _worker_id()

    _fill(rows0, KE, 0.0)
    _zero_acc_slice(rows0, acc_sh, sid)

    pltpu.sync_copy(src_hbm.at[wid], src_v)
    pltpu.sync_copy(dst_hbm.at[wid, pl.ds(0, WPB)], dwa)
    pltpu.async_copy(dst_hbm.at[wid, pl.ds(WPB, WPB)], dwb, wsb)
    plsc.subcore_barrier()

    def _gather(i, buf, sem):
        pltpu.async_copy(h_hbm.at[src_v.at[i]], buf, sem)

    def _gwait(i, buf, sem):
        pltpu.make_async_copy(h_hbm.at[src_v.at[i]], buf, sem).wait()

    def _win_start(w, nch, buf, sem):
        pltpu.async_copy(
            dst_hbm.at[wid, pl.ds(w * WPB, nch)], buf.at[pl.ds(0, nch)], sem
        )

    def _win_wait(w, nch, buf, sem):
        pltpu.make_async_copy(
            dst_hbm.at[wid, pl.ds(w * WPB, nch)], buf.at[pl.ds(0, nch)], sem
        ).wait()

    def _chunk_step(i, j, win_buf):
        buf, sem = (rows0, gs0) if j % 2 == 0 else (rows1, gs1)
        _gwait(i, buf, sem)
        pltpu.sync_copy(buf, acc_sh.at[win_buf.at[j]], add=True)

        @pl.when(i + 2 < NCHP)
        def _():
            _gather(i + 2, buf, sem)

    _gather(0, rows0, gs0)
    _gather(1, rows1, gs1)

    @pl.loop(0, FW - 1, step=2)
    def _(w):
        base = w * WPB
        for j in range(WPB):
            _chunk_step(base + j, j, dwa)

        @pl.when(w + 2 < FW)
        def _():
            _win_start(w + 2, WPB, dwa, wsa)

        _win_wait(w + 1, WPB, dwb, wsb)
        for j in range(WPB):
            _chunk_step(base + WPB + j, j, dwb)

        @pl.when(w + 3 < FW)
        def _():
            _win_start(w + 3, WPB, dwb, wsb)

        @pl.when(w + 2 < FW)
        def _():
            _win_wait(w + 2, WPB, dwa, wsa)

    pltpu.async_copy(
        dst_hbm.at[wid, pl.ds(FW * WPB, TWC)], dwb.at[pl.ds(0, TWC)], wsb
    )
    for j in range(WPB):
        _chunk_step((FW - 1) * WPB + j, j, dwa)

    pltpu.make_async_copy(
        dst_hbm.at[wid, pl.ds(FW * WPB, TWC)], dwb.at[pl.ds(0, TWC)], wsb
    ).wait()
    for j in range(TWC):
        _chunk_step(FW * WPB + j, j, dwb)

    plsc.subcore_barrier()
    pltpu.sync_copy(
        acc_sh.at[pl.ds(sid * RPS, RPS)],
        out_hbm.at[cid, sid],
    )


_RB = 400
_GRID = N // _RB

_row_spec = pl.BlockSpec((_RB, D), lambda i: (i, 0))
_pair_spec = pl.BlockSpec((NC, _RB, D), lambda i: (0, i, 0))
_w_spec = pl.BlockSpec((D, D), lambda i: (0, 0))
_b_spec = pl.BlockSpec((1, D), lambda i: (0, 0))


def _dinv_of(hist_ref):
    deg = hist_ref[0, :, 0] + hist_ref[1, :, 0] + 1.0
    return lax.rsqrt(deg)


def _prep_body(x_ref, w_ref, hist_ref, o_ref):
    dinv = _dinv_of(hist_ref)
    h = jnp.dot(x_ref[...], w_ref[...], preferred_element_type=jnp.float32)
    o_ref[...] = h * dinv[:, None]


_prep = pl.pallas_call(
    _prep_body,
    grid=(_GRID,),
    in_specs=[_row_spec, _w_spec, _pair_spec],
    out_specs=_row_spec,
    out_shape=jax.ShapeDtypeStruct((N, D), jnp.float32),
)


def _mid_body(p_ref, hp_ref, b_ref, w_ref, hist_ref, o_ref):
    dinv = _dinv_of(hist_ref)
    pre = dinv[:, None] * (p_ref[0] + p_ref[1] + hp_ref[...]) + b_ref[...]
    act = jnp.maximum(pre, 0.0)
    h = jnp.dot(act, w_ref[...], preferred_element_type=jnp.float32)
    o_ref[...] = h * dinv[:, None]


_mid = pl.pallas_call(
    _mid_body,
    grid=(_GRID,),
    in_specs=[_pair_spec, _row_spec, _b_spec, _w_spec, _pair_spec],
    out_specs=_row_spec,
    out_shape=jax.ShapeDtypeStruct((N, D), jnp.float32),
)


def _final_body(p_ref, hp_ref, b_ref, hist_ref, o_ref):
    dinv = _dinv_of(hist_ref)
    o_ref[...] = (
        dinv[:, None] * (p_ref[0] + p_ref[1] + hp_ref[...]) + b_ref[...]
    )


_final = pl.pallas_call(
    _final_body,
    grid=(_GRID,),
    in_specs=[_pair_spec, _row_spec, _b_spec, _pair_spec],
    out_specs=_row_spec,
    out_shape=jax.ShapeDtypeStruct((N, D), jnp.float32),
)


def kernel(x, edge_index, W1, b1, W2, b2):
    src = edge_index[0].astype(jnp.int32).reshape(NW, NCHP, KE)
    dst = edge_index[1].astype(jnp.int32).reshape(NW, NCHP, KE)
    b1r = b1.reshape(1, D)
    b2r = b2.reshape(1, D)

    hist = _hist_kernel(dst).reshape(NC, N, D)

    h1p = _prep(x, W1, hist)
    p = _agg_kernel(h1p, src, dst).reshape(NC, N, D)
    h2p = _mid(p, h1p, b1r, W2, hist)
    q = _agg_kernel(h2p, src, dst).reshape(NC, N, D)
    out = _final(q, h2p, b2r, hist)
    return out
